# Initial kernel scaffold; baseline (speedup 1.0000x reference)
import jax
import jax.numpy as jnp
from jax.experimental import pallas as pl


def kernel(feat_list, edge_index, n_step, W_gcn, b_gcn, Wi, Wh, bi, bh, W1, b1, W2, b2):
    raise NotImplementedError("write your pallas kernel here")


# speedup vs baseline: 9.9884x; 9.9884x over previous
import functools

import jax
import jax.numpy as jnp
from jax import lax
from jax.experimental import pallas as pl
from jax.experimental.pallas import tpu as pltpu
from jax.experimental.pallas import tpu_sc as plsc

NC = 2
NS = 16
NW = NC * NS
CHUNK = 128
FH = 64


def _round_up(x, m):
    return (x + m - 1) // m * m


def _sc_degree(dst_r, *, npad, nch):
    mesh = plsc.VectorSubcoreMesh(core_axis_name="c", subcore_axis_name="s")
    rpt = npad // NS
    nstrip = rpt // CHUNK

    @functools.partial(
        pl.kernel,
        out_type=jax.ShapeDtypeStruct((NC, npad, 16), jnp.float32),
        mesh=mesh,
        scratch_types=[
            pltpu.VMEM((nch, CHUNK), jnp.int32),
            pltpu.VMEM((CHUNK, 16), jnp.float32),
            pltpu.VMEM_SHARED((npad, 16), jnp.float32),
        ],
        compiler_params=pltpu.CompilerParams(use_tc_tiling_on_sc=False),
    )
    def deg_kernel(dst_hbm, out_hbm, dst_v, strip_v, deg_sh):
        cid = lax.axis_index("c")
        sid = lax.axis_index("s")
        wid = sid * NC + cid
        base = sid * rpt
        pltpu.sync_copy(dst_hbm.at[wid], dst_v)

        def fill(val):
            def body(j, carry):
                strip_v[j] = jnp.full((16,), val, jnp.float32)
                return carry
            lax.fori_loop(0, CHUNK, body, 0)

        fill(0.0)

        def zcopy(k, carry):
            pltpu.sync_copy(strip_v, deg_sh.at[pl.ds(base + k * CHUNK, CHUNK)])
            return carry
        lax.fori_loop(0, nstrip, zcopy, 0)
        fill(1.0)
        plsc.subcore_barrier()

        def body(i, carry):
            pltpu.sync_copy(strip_v, deg_sh.at[dst_v.at[i]], add=True)
            return carry
        lax.fori_loop(0, nch, body, 0)
        plsc.subcore_barrier()

        def out_copy(k, carry):
            sl = pl.ds(base + k * CHUNK, CHUNK)
            pltpu.sync_copy(deg_sh.at[sl], strip_v)
            pltpu.sync_copy(strip_v, out_hbm.at[cid, sl])
            return carry
        lax.fori_loop(0, nstrip, out_copy, 0)

    return deg_kernel(dst_r)


def _sc_scatter(xs0, xs1, src_r, dst_r, *, npad, nch):
    mesh = plsc.VectorSubcoreMesh(core_axis_name="c", subcore_axis_name="s")
    rpt = npad // NS
    nstrip = rpt // CHUNK

    @functools.partial(
        pl.kernel,
        out_type=jax.ShapeDtypeStruct((2, NC, npad, FH), jnp.float32),
        mesh=mesh,
        scratch_types=[
            pltpu.VMEM((nch, CHUNK), jnp.int32),
            pltpu.VMEM((nch, CHUNK), jnp.int32),
            pltpu.VMEM((CHUNK, FH), jnp.float32),
            pltpu.VMEM((CHUNK, FH), jnp.float32),
            pltpu.VMEM_SHARED((npad, FH), jnp.float32),
            pltpu.SemaphoreType.DMA,
            pltpu.SemaphoreType.DMA,
        ],
        compiler_params=pltpu.CompilerParams(use_tc_tiling_on_sc=False),
    )
    def scat_kernel(xs0_hbm, xs1_hbm, src_hbm, dst_hbm, out_hbm,
                    src_v, dst_v, rows0_v, rows1_v, agg_sh, sem0, sem1):
        cid = lax.axis_index("c")
        sid = lax.axis_index("s")
        wid = sid * NC + cid
        base = sid * rpt
        pltpu.sync_copy(src_hbm.at[wid], src_v)
        pltpu.sync_copy(dst_hbm.at[wid], dst_v)

        for f in range(2):
            xs_hbm = (xs0_hbm, xs1_hbm)[f]
            def zfill(j, carry):
                r = j // (FH // 16)
                col = j % (FH // 16)
                rows0_v[r, pl.ds(col * 16, 16)] = jnp.zeros((16,), jnp.float32)
                return carry
            lax.fori_loop(0, CHUNK * (FH // 16), zfill, 0)

            def zcopy(k, carry):
                pltpu.sync_copy(rows0_v,
                                agg_sh.at[pl.ds(base + k * CHUNK, CHUNK)])
                return carry
            lax.fori_loop(0, nstrip, zcopy, 0)
            plsc.subcore_barrier()

            pltpu.async_copy(xs_hbm.at[src_v.at[0]], rows0_v, sem0)
            pltpu.async_copy(xs_hbm.at[src_v.at[1]], rows1_v, sem1)

            def body(i, carry):
                pltpu.make_async_copy(
                    xs_hbm.at[src_v.at[i]], rows0_v, sem0).wait()
                pltpu.sync_copy(rows0_v, agg_sh.at[dst_v.at[i]], add=True)
                nxt0 = lax.rem(i + 2, nch)
                pltpu.async_copy(xs_hbm.at[src_v.at[nxt0]], rows0_v, sem0)
                pltpu.make_async_copy(
                    xs_hbm.at[src_v.at[i + 1]], rows1_v, sem1).wait()
                pltpu.sync_copy(rows1_v, agg_sh.at[dst_v.at[i + 1]], add=True)
                nxt1 = lax.rem(i + 3, nch)
                pltpu.async_copy(xs_hbm.at[src_v.at[nxt1]], rows1_v, sem1)
                return carry
            lax.fori_loop(0, nch // 2, lambda k, c: body(2 * k, c), 0)
            pltpu.make_async_copy(xs_hbm.at[src_v.at[0]], rows0_v, sem0).wait()
            pltpu.make_async_copy(xs_hbm.at[src_v.at[0]], rows1_v, sem1).wait()
            plsc.subcore_barrier()

            def out_copy(k, carry):
                sl = pl.ds(base + k * CHUNK, CHUNK)
                pltpu.sync_copy(agg_sh.at[sl], rows0_v)
                pltpu.sync_copy(rows0_v, out_hbm.at[f, cid, sl])
                return carry
            lax.fori_loop(0, nstrip, out_copy, 0)

    return scat_kernel(xs0, xs1, src_r, dst_r)


def _proj_body(f_ref, w_ref, b_ref, d0_ref, d1_ref,
               xs0_ref, xs1_ref, norm_ref):
    deg = d0_ref[0][:, 0:1] + d1_ref[0][:, 0:1]
    nrm = lax.rsqrt(jnp.clip(deg, 1.0, None))
    x = jnp.dot(f_ref[0], w_ref[...], preferred_element_type=jnp.float32)
    xs = (x + b_ref[...]) * nrm
    xs0_ref[0] = xs[:, 0:FH]
    xs1_ref[0] = xs[:, FH:2 * FH]
    norm_ref[...] = nrm


def _tc_project(feat, w, b2d, degp, *, br):
    t, n, din = feat.shape
    hid = w.shape[1]
    nblk = n // br
    return pl.pallas_call(
        _proj_body,
        grid=(t, nblk),
        in_specs=[
            pl.BlockSpec((1, br, din), lambda i, r: (i, r, 0)),
            pl.BlockSpec((din, hid), lambda i, r: (0, 0)),
            pl.BlockSpec((1, hid), lambda i, r: (0, 0)),
            pl.BlockSpec((1, br, 16), lambda i, r: (0, r, 0)),
            pl.BlockSpec((1, br, 16), lambda i, r: (1, r, 0)),
        ],
        out_specs=[
            pl.BlockSpec((1, br, FH), lambda i, r: (i, r, 0)),
            pl.BlockSpec((1, br, FH), lambda i, r: (i, r, 0)),
            pl.BlockSpec((br, 1), lambda i, r: (r, 0)),
        ],
        out_shape=[
            jax.ShapeDtypeStruct((t, n, FH), jnp.float32),
            jax.ShapeDtypeStruct((t, n, FH), jnp.float32),
            jax.ShapeDtypeStruct((n, 1), jnp.float32),
        ],
    )(feat, w, b2d, degp, degp)


def _lstm_body(p00_ref, p01_ref, p10_ref, p11_ref, nrm_ref, h_ref, c_ref,
               wi_ref, wh_ref, bs_ref, h2_ref, c2_ref):
    hid = h_ref.shape[-1]
    agg = jnp.concatenate(
        [p00_ref[0, 0] + p01_ref[0, 0], p10_ref[0, 0] + p11_ref[0, 0]],
        axis=1)
    g = jnp.maximum(agg * nrm_ref[...], 0.0)
    gates = (jnp.dot(g, wi_ref[...], preferred_element_type=jnp.float32)
             + jnp.dot(h_ref[...], wh_ref[...],
                       preferred_element_type=jnp.float32)
             + bs_ref[...])
    i_g = jax.nn.sigmoid(gates[:, 0:hid])
    f_g = jax.nn.sigmoid(gates[:, hid:2 * hid])
    g_g = jnp.tanh(gates[:, 2 * hid:3 * hid])
    o_g = jax.nn.sigmoid(gates[:, 3 * hid:4 * hid])
    cn = f_g * c_ref[...] + i_g * g_g
    h2_ref[...] = o_g * jnp.tanh(cn)
    c2_ref[...] = cn


def _tc_lstm(part, nrm, h, c, wi_t, wh_t, bsum, *, br):
    n, hid = h.shape
    nblk = n // br
    pspec = lambda f, cc: pl.BlockSpec(
        (1, 1, br, FH), lambda r, _f=f, _c=cc: (_f, _c, r, 0))
    return pl.pallas_call(
        _lstm_body,
        grid=(nblk,),
        in_specs=[
            pspec(0, 0), pspec(0, 1), pspec(1, 0), pspec(1, 1),
            pl.BlockSpec((br, 1), lambda r: (r, 0)),
            pl.BlockSpec((br, hid), lambda r: (r, 0)),
            pl.BlockSpec((br, hid), lambda r: (r, 0)),
            pl.BlockSpec((hid, 4 * hid), lambda r: (0, 0)),
            pl.BlockSpec((hid, 4 * hid), lambda r: (0, 0)),
            pl.BlockSpec((1, 4 * hid), lambda r: (0, 0)),
        ],
        out_specs=[
            pl.BlockSpec((br, hid), lambda r: (r, 0)),
            pl.BlockSpec((br, hid), lambda r: (r, 0)),
        ],
        out_shape=[
            jax.ShapeDtypeStruct((n, hid), jnp.float32),
            jax.ShapeDtypeStruct((n, hid), jnp.float32),
        ],
    )(part, part, part, part, nrm, h, c, wi_t, wh_t, bsum)


def _mlp_body(h_ref, w1_ref, b1_ref, w2_ref, b2_ref, o_ref):
    z = jnp.maximum(
        jnp.dot(h_ref[...], w1_ref[...], preferred_element_type=jnp.float32)
        + b1_ref[...], 0.0)
    o_ref[...] = (jnp.dot(z, w2_ref[...], preferred_element_type=jnp.float32)
                  + b2_ref[...])


def _tc_mlp(h, w1, b1_2d, w2, b2_2d, *, br):
    n, hid = h.shape
    cls = w1.shape[1]
    dout = w2.shape[1]
    nblk = n // br
    return pl.pallas_call(
        _mlp_body,
        grid=(nblk,),
        in_specs=[
            pl.BlockSpec((br, hid), lambda r: (r, 0)),
            pl.BlockSpec((hid, cls), lambda r: (0, 0)),
            pl.BlockSpec((1, cls), lambda r: (0, 0)),
            pl.BlockSpec((cls, dout), lambda r: (0, 0)),
            pl.BlockSpec((1, dout), lambda r: (0, 0)),
        ],
        out_specs=pl.BlockSpec((br, dout), lambda r: (r, 0)),
        out_shape=jax.ShapeDtypeStruct((n, dout), jnp.float32),
    )(h, w1, b1_2d, w2, b2_2d)


def kernel(feat_list, edge_index, n_step, W_gcn, b_gcn, Wi, Wh, bi, bh,
           W1, b1, W2, b2):
    del n_step
    t_steps, n, _ = feat_list.shape
    hid = W_gcn.shape[1]
    br = 1000 if n % 1000 == 0 else 8 * (n // 8)

    src = edge_index[0]
    dst = edge_index[1]
    e = src.shape[0]

    npad = _round_up(n + 1, NS * CHUNK)
    epad = _round_up(e, NW * CHUNK * 2)
    nch = epad // (NW * CHUNK)
    pad = epad - e
    pad_i = jnp.arange(pad, dtype=jnp.int32)
    src_p = jnp.concatenate([src, pad_i % n])
    dst_p = jnp.concatenate([dst, n + pad_i % (npad - n)])
    src_r = src_p.reshape(NW, nch, CHUNK)
    dst_r = dst_p.reshape(NW, nch, CHUNK)

    degp = _sc_degree(dst_r, npad=npad, nch=nch)

    b2d = b_gcn.reshape(1, hid)
    xs0, xs1, nrm = _tc_project(feat_list, W_gcn, b2d, degp, br=br)

    wi_t = Wi.T
    wh_t = Wh.T
    bsum = (bi + bh).reshape(1, 4 * hid)
    h = jnp.zeros((n, hid), jnp.float32)
    c = jnp.zeros((n, hid), jnp.float32)
    for t in range(t_steps):
        part = _sc_scatter(xs0[t], xs1[t], src_r, dst_r, npad=npad, nch=nch)
        h, c = _tc_lstm(part, nrm, h, c, wi_t, wh_t, bsum, br=br)

    return _tc_mlp(h, W1, b1.reshape(1, -1), W2, b2.reshape(1, -1), br=br)


# speedup vs baseline: 11.4126x; 1.1426x over previous
import functools

import jax
import jax.numpy as jnp
from jax import lax
from jax.experimental import pallas as pl
from jax.experimental.pallas import tpu as pltpu
from jax.experimental.pallas import tpu_sc as plsc

NC = 2
NS = 16
CHUNK = 128
FH = 64
NSLOT = 4
LOOK = 2


def _round_up(x, m):
    return (x + m - 1) // m * m


def _sc_degree(dst_r, *, npad, nch):
    mesh = plsc.VectorSubcoreMesh(core_axis_name="c", subcore_axis_name="s")
    rpt = npad // NS
    nstrip = rpt // CHUNK
    nchc = nch // NC

    @functools.partial(
        pl.kernel,
        out_type=jax.ShapeDtypeStruct((NC, npad, 16), jnp.float32),
        mesh=mesh,
        scratch_types=[
            pltpu.VMEM((nchc, CHUNK), jnp.int32),
            pltpu.VMEM((CHUNK, 16), jnp.float32),
            pltpu.VMEM((CHUNK, 16), jnp.float32),
            pltpu.VMEM_SHARED((npad, 16), jnp.float32),
            pltpu.SemaphoreType.DMA,
        ],
        compiler_params=pltpu.CompilerParams(use_tc_tiling_on_sc=False),
    )
    def deg_kernel(dst_hbm, out_hbm, dst_v, ones_v, zero_v, deg_sh, sem):
        cid = lax.axis_index("c")
        sid = lax.axis_index("s")
        base = sid * rpt
        pltpu.sync_copy(dst_hbm.at[sid, pl.ds(cid * nchc, nchc)], dst_v)

        def fill(buf, val):
            def body(j, carry):
                buf[j] = jnp.full((16,), val, jnp.float32)
                return carry
            lax.fori_loop(0, CHUNK, body, 0)

        fill(zero_v, 0.0)
        fill(ones_v, 1.0)

        def zcopy(k, carry):
            pltpu.sync_copy(zero_v, deg_sh.at[pl.ds(base + k * CHUNK, CHUNK)])
            return carry
        lax.fori_loop(0, nstrip, zcopy, 0)
        plsc.subcore_barrier()

        def round8(r, carry):
            for j in range(8):
                pltpu.async_copy(ones_v, deg_sh.at[dst_v.at[r * 8 + j]], sem,
                                 add=True)
            for j in range(8):
                pltpu.make_async_copy(
                    ones_v, deg_sh.at[dst_v.at[r * 8 + j]], sem).wait()
            return carry
        lax.fori_loop(0, nchc // 8, round8, 0)
        plsc.subcore_barrier()

        def out_copy(k, carry):
            sl = pl.ds(base + k * CHUNK, CHUNK)
            pltpu.sync_copy(deg_sh.at[sl], ones_v)
            pltpu.sync_copy(ones_v, out_hbm.at[cid, sl])
            return carry
        lax.fori_loop(0, nstrip, out_copy, 0)

    return deg_kernel(dst_r)


def _sc_scatter(xs_all, src_r, dst_r, dep, *, t, npad, nch):
    mesh = plsc.VectorSubcoreMesh(core_axis_name="c", subcore_axis_name="s")
    rpt = npad // NS
    nstrip = rpt // CHUNK

    @functools.partial(
        pl.kernel,
        out_type=jax.ShapeDtypeStruct((NC, npad, FH), jnp.float32),
        mesh=mesh,
        scratch_types=[
            pltpu.VMEM((nch, CHUNK), jnp.int32),
            pltpu.VMEM((nch, CHUNK), jnp.int32),
        ] + [pltpu.VMEM((CHUNK, FH), jnp.float32) for _ in range(NSLOT)] + [
            pltpu.VMEM_SHARED((npad, FH), jnp.float32),
        ] + [pltpu.SemaphoreType.DMA for _ in range(2 * NSLOT)],
        compiler_params=pltpu.CompilerParams(use_tc_tiling_on_sc=False),
    )
    def scat_kernel(xs_hbm, src_hbm, dst_hbm, dep_hbm, out_hbm,
                    src_v, dst_v, *rest):
        del dep_hbm
        bufs = rest[:NSLOT]
        agg_sh = rest[NSLOT]
        gsem = rest[NSLOT + 1:NSLOT + 1 + NSLOT]
        ssem = rest[NSLOT + 1 + NSLOT:]
        cid = lax.axis_index("c")
        sid = lax.axis_index("s")
        base = sid * rpt
        my_xs = xs_hbm.at[t, cid]

        pltpu.sync_copy(src_hbm.at[sid], src_v)
        pltpu.sync_copy(dst_hbm.at[sid], dst_v)

        def gath(k, slot):
            pltpu.async_copy(my_xs.at[src_v.at[k]], bufs[slot], gsem[slot])

        def gwait(k, slot):
            pltpu.make_async_copy(
                my_xs.at[src_v.at[k]], bufs[slot], gsem[slot]).wait()

        def scat(k, slot):
            pltpu.async_copy(bufs[slot], agg_sh.at[dst_v.at[k]], ssem[slot],
                             add=True)

        def swait(k, slot):
            pltpu.make_async_copy(
                bufs[slot], agg_sh.at[dst_v.at[k]], ssem[slot]).wait()

        for k in range(LOOK):
            gath(k, k)

        zbuf = bufs[NSLOT - 1]

        def zfill(j, carry):
            r = j // (FH // 16)
            col = j % (FH // 16)
            zbuf[r, pl.ds(col * 16, 16)] = jnp.zeros((16,), jnp.float32)
            return carry
        lax.fori_loop(0, CHUNK * (FH // 16), zfill, 0)

        def zcopy(k, carry):
            pltpu.sync_copy(zbuf, agg_sh.at[pl.ds(base + k * CHUNK, CHUNK)])
            return carry
        lax.fori_loop(0, nstrip, zcopy, 0)
        plsc.subcore_barrier()

        for k in range(LOOK):
            gwait(k, k)
            scat(k, k)
            gath(k + LOOK, k + LOOK)

        def step(k, slot):
            gwait(k, slot)
            scat(k, slot)
            old = (slot + LOOK) % NSLOT
            swait(k - LOOK, old)
            gath(k + LOOK, old)

        def ring(i, carry):
            kb = LOOK + i * NSLOT
            for j in range(NSLOT):
                step(kb + j, (LOOK + j) % NSLOT)
            return carry
        lax.fori_loop(0, (nch - 2 * LOOK) // NSLOT, ring, 0)

        for k in range(nch - LOOK, nch):
            slot = k % NSLOT
            gwait(k, slot)
            scat(k, slot)
            swait(k - LOOK, (slot + LOOK) % NSLOT)
        for k in range(nch - LOOK, nch):
            swait(k, k % NSLOT)
        plsc.subcore_barrier()

        def out_copy(k, carry):
            sl = pl.ds(base + k * CHUNK, CHUNK)
            pltpu.sync_copy(agg_sh.at[sl], bufs[0])
            pltpu.sync_copy(bufs[0], out_hbm.at[cid, sl])
            return carry
        lax.fori_loop(0, nstrip, out_copy, 0)

    return scat_kernel(xs_all, src_r, dst_r, dep)


def _proj_first_body(f_ref, w_ref, b_ref, d0_ref, d1_ref, xs_ref, norm_ref):
    deg = d0_ref[0][:, 0:1] + d1_ref[0][:, 0:1]
    nrm = lax.rsqrt(jnp.clip(deg, 1.0, None))
    x = jnp.dot(f_ref[0], w_ref[...], preferred_element_type=jnp.float32)
    xs = (x + b_ref[...]) * nrm
    xs_ref[0, 0] = xs[:, 0:FH]
    xs_ref[0, 1] = xs[:, FH:2 * FH]
    norm_ref[...] = nrm


def _tc_project_first(feat, w, b2d, degp, *, br):
    t, n, din = feat.shape
    hid = w.shape[1]
    nblk = n // br
    return pl.pallas_call(
        _proj_first_body,
        grid=(t, nblk),
        in_specs=[
            pl.BlockSpec((1, br, din), lambda i, r: (i, r, 0)),
            pl.BlockSpec((din, hid), lambda i, r: (0, 0)),
            pl.BlockSpec((1, hid), lambda i, r: (0, 0)),
            pl.BlockSpec((1, br, 16), lambda i, r: (0, r, 0)),
            pl.BlockSpec((1, br, 16), lambda i, r: (1, r, 0)),
        ],
        out_specs=[
            pl.BlockSpec((1, 2, br, FH), lambda i, r: (i, 0, r, 0)),
            pl.BlockSpec((br, 1), lambda i, r: (r, 0)),
        ],
        out_shape=[
            jax.ShapeDtypeStruct((t, 2, n, FH), jnp.float32),
            jax.ShapeDtypeStruct((n, 1), jnp.float32),
        ],
    )(feat, w, b2d, degp, degp)


def _proj_rest_body(f_ref, w_ref, b_ref, nrm_ref, xs_ref):
    x = jnp.dot(f_ref[0], w_ref[...], preferred_element_type=jnp.float32)
    xs = (x + b_ref[...]) * nrm_ref[...]
    xs_ref[0, 0] = xs[:, 0:FH]
    xs_ref[0, 1] = xs[:, FH:2 * FH]


def _tc_project_rest(feat, w, b2d, nrm, *, br):
    t, n, din = feat.shape
    hid = w.shape[1]
    nblk = n // br
    return pl.pallas_call(
        _proj_rest_body,
        grid=(t, nblk),
        in_specs=[
            pl.BlockSpec((1, br, din), lambda i, r: (i, r, 0)),
            pl.BlockSpec((din, hid), lambda i, r: (0, 0)),
            pl.BlockSpec((1, hid), lambda i, r: (0, 0)),
            pl.BlockSpec((br, 1), lambda i, r: (r, 0)),
        ],
        out_specs=pl.BlockSpec((1, 2, br, FH), lambda i, r: (i, 0, r, 0)),
        out_shape=jax.ShapeDtypeStruct((t, 2, n, FH), jnp.float32),
    )(feat, w, b2d, nrm)


def _lstm_body(p0_ref, p1_ref, nrm_ref, h_ref, c_ref,
               wi_ref, wh_ref, bs_ref, h2_ref, c2_ref):
    hid = h_ref.shape[-1]
    agg = jnp.concatenate([p0_ref[0], p1_ref[0]], axis=1)
    g = jnp.maximum(agg * nrm_ref[...], 0.0)
    gates = (jnp.dot(g, wi_ref[...], preferred_element_type=jnp.float32)
             + jnp.dot(h_ref[...], wh_ref[...],
                       preferred_element_type=jnp.float32)
             + bs_ref[...])
    i_g = jax.nn.sigmoid(gates[:, 0:hid])
    f_g = jax.nn.sigmoid(gates[:, hid:2 * hid])
    g_g = jnp.tanh(gates[:, 2 * hid:3 * hid])
    o_g = jax.nn.sigmoid(gates[:, 3 * hid:4 * hid])
    cn = f_g * c_ref[...] + i_g * g_g
    h2_ref[...] = o_g * jnp.tanh(cn)
    c2_ref[...] = cn


def _tc_lstm(part, nrm, h, c, wi_t, wh_t, bsum, *, br):
    n, hid = h.shape
    nblk = n // br
    return pl.pallas_call(
        _lstm_body,
        grid=(nblk,),
        in_specs=[
            pl.BlockSpec((1, br, FH), lambda r: (0, r, 0)),
            pl.BlockSpec((1, br, FH), lambda r: (1, r, 0)),
            pl.BlockSpec((br, 1), lambda r: (r, 0)),
            pl.BlockSpec((br, hid), lambda r: (r, 0)),
            pl.BlockSpec((br, hid), lambda r: (r, 0)),
            pl.BlockSpec((hid, 4 * hid), lambda r: (0, 0)),
            pl.BlockSpec((hid, 4 * hid), lambda r: (0, 0)),
            pl.BlockSpec((1, 4 * hid), lambda r: (0, 0)),
        ],
        out_specs=[
            pl.BlockSpec((br, hid), lambda r: (r, 0)),
            pl.BlockSpec((br, hid), lambda r: (r, 0)),
        ],
        out_shape=[
            jax.ShapeDtypeStruct((n, hid), jnp.float32),
            jax.ShapeDtypeStruct((n, hid), jnp.float32),
        ],
    )(part, part, nrm, h, c, wi_t, wh_t, bsum)


def _mlp_body(h_ref, w1_ref, b1_ref, w2_ref, b2_ref, o_ref):
    z = jnp.maximum(
        jnp.dot(h_ref[...], w1_ref[...], preferred_element_type=jnp.float32)
        + b1_ref[...], 0.0)
    o_ref[...] = (jnp.dot(z, w2_ref[...], preferred_element_type=jnp.float32)
                  + b2_ref[...])


def _tc_mlp(h, w1, b1_2d, w2, b2_2d, *, br):
    n, hid = h.shape
    cls = w1.shape[1]
    dout = w2.shape[1]
    nblk = n // br
    return pl.pallas_call(
        _mlp_body,
        grid=(nblk,),
        in_specs=[
            pl.BlockSpec((br, hid), lambda r: (r, 0)),
            pl.BlockSpec((hid, cls), lambda r: (0, 0)),
            pl.BlockSpec((1, cls), lambda r: (0, 0)),
            pl.BlockSpec((cls, dout), lambda r: (0, 0)),
            pl.BlockSpec((1, dout), lambda r: (0, 0)),
        ],
        out_specs=pl.BlockSpec((br, dout), lambda r: (r, 0)),
        out_shape=jax.ShapeDtypeStruct((n, dout), jnp.float32),
    )(h, w1, b1_2d, w2, b2_2d)


def kernel(feat_list, edge_index, n_step, W_gcn, b_gcn, Wi, Wh, bi, bh,
           W1, b1, W2, b2):
    del n_step
    t_steps, n, _ = feat_list.shape
    hid = W_gcn.shape[1]
    br = 1000 if n % 1000 == 0 else 8 * (n // 8)

    src = edge_index[0]
    dst = edge_index[1]
    e = src.shape[0]

    npad = _round_up(n + 1, NS * CHUNK)
    epad = _round_up(e, NS * CHUNK * NSLOT * NC)
    nch = epad // (NS * CHUNK)
    pad = epad - e
    pad_i = jnp.arange(pad, dtype=jnp.int32)
    src_p = jnp.concatenate([src, pad_i % n])
    dst_p = jnp.concatenate([dst, n + pad_i % (npad - n)])
    src_r = src_p.reshape(NS, nch, CHUNK)
    dst_r = dst_p.reshape(NS, nch, CHUNK)

    degp = _sc_degree(dst_r, npad=npad, nch=nch)

    b2d = b_gcn.reshape(1, hid)
    xs_t0, nrm = _tc_project_first(feat_list[0:1], W_gcn, b2d, degp, br=br)
    xs_rest = _tc_project_rest(feat_list[1:], W_gcn, b2d, nrm, br=br)

    wi_t = Wi.T
    wh_t = Wh.T
    bsum = (bi + bh).reshape(1, 4 * hid)
    h = jnp.zeros((n, hid), jnp.float32)
    c = jnp.zeros((n, hid), jnp.float32)
    hl = [h]
    for t in range(t_steps):
        xs_arr, tt = (xs_t0, 0) if t == 0 else (xs_rest, t - 1)
        part = _sc_scatter(xs_arr, src_r, dst_r, hl[max(t - 1, 0)],
                           t=tt, npad=npad, nch=nch)
        h, c = _tc_lstm(part, nrm, h, c, wi_t, wh_t, bsum, br=br)
        hl.append(h)

    return _tc_mlp(h, W1, b1.reshape(1, -1), W2, b2.reshape(1, -1), br=br)


# speedup vs baseline: 11.5412x; 1.0113x over previous
import functools

import jax
import jax.numpy as jnp
from jax import lax
from jax.experimental import pallas as pl
from jax.experimental.pallas import tpu as pltpu
from jax.experimental.pallas import tpu_sc as plsc

NC = 2
NS = 16
CHUNK = 128
FH = 64
NSLOT = 4
LOOK = 2


def _round_up(x, m):
    return (x + m - 1) // m * m


def _sc_degree(dst_r, *, npad, nch):
    mesh = plsc.VectorSubcoreMesh(core_axis_name="c", subcore_axis_name="s")
    rpt = npad // NS
    nstrip = rpt // CHUNK
    nchc = nch // NC

    @functools.partial(
        pl.kernel,
        out_type=jax.ShapeDtypeStruct((NC, npad, 16), jnp.float32),
        mesh=mesh,
        scratch_types=[
            pltpu.VMEM((nchc, CHUNK), jnp.int32),
            pltpu.VMEM((CHUNK, 16), jnp.float32),
            pltpu.VMEM((CHUNK, 16), jnp.float32),
            pltpu.VMEM_SHARED((npad, 16), jnp.float32),
            pltpu.SemaphoreType.DMA,
        ],
        compiler_params=pltpu.CompilerParams(use_tc_tiling_on_sc=False),
    )
    def deg_kernel(dst_hbm, out_hbm, dst_v, ones_v, zero_v, deg_sh, sem):
        cid = lax.axis_index("c")
        sid = lax.axis_index("s")
        base = sid * rpt
        pltpu.sync_copy(dst_hbm.at[sid, pl.ds(cid * nchc, nchc)], dst_v)

        def fill(buf, val):
            def body(j, carry):
                buf[j] = jnp.full((16,), val, jnp.float32)
                return carry
            lax.fori_loop(0, CHUNK, body, 0)

        fill(zero_v, 0.0)
        fill(ones_v, 1.0)

        def zcopy(k, carry):
            pltpu.sync_copy(zero_v, deg_sh.at[pl.ds(base + k * CHUNK, CHUNK)])
            return carry
        lax.fori_loop(0, nstrip, zcopy, 0)
        plsc.subcore_barrier()

        def round8(r, carry):
            for j in range(8):
                pltpu.async_copy(ones_v, deg_sh.at[dst_v.at[r * 8 + j]], sem,
                                 add=True)
            for j in range(8):
                pltpu.make_async_copy(
                    ones_v, deg_sh.at[dst_v.at[r * 8 + j]], sem).wait()
            return carry
        lax.fori_loop(0, nchc // 8, round8, 0)
        plsc.subcore_barrier()

        def out_copy(k, carry):
            sl = pl.ds(base + k * CHUNK, CHUNK)
            pltpu.sync_copy(deg_sh.at[sl], ones_v)
            pltpu.sync_copy(ones_v, out_hbm.at[cid, sl])
            return carry
        lax.fori_loop(0, nstrip, out_copy, 0)

    return deg_kernel(dst_r)


def _sc_scatter(xs_all, src_r, dst_r, *, t, npad, nch):
    mesh = plsc.VectorSubcoreMesh(core_axis_name="c", subcore_axis_name="s")
    rpt = npad // NS
    nstrip = rpt // CHUNK

    @functools.partial(
        pl.kernel,
        out_type=jax.ShapeDtypeStruct((NC, npad, FH), jnp.float32),
        mesh=mesh,
        scratch_types=[
            pltpu.VMEM((nch, CHUNK), jnp.int32),
            pltpu.VMEM((nch, CHUNK), jnp.int32),
        ] + [pltpu.VMEM((CHUNK, FH), jnp.float32) for _ in range(NSLOT)] + [
            pltpu.VMEM_SHARED((npad, FH), jnp.float32),
        ] + [pltpu.SemaphoreType.DMA for _ in range(2 * NSLOT)],
        compiler_params=pltpu.CompilerParams(use_tc_tiling_on_sc=False),
    )
    def scat_kernel(xs_hbm, src_hbm, dst_hbm, out_hbm,
                    src_v, dst_v, *rest):
        bufs = rest[:NSLOT]
        agg_sh = rest[NSLOT]
        gsem = rest[NSLOT + 1:NSLOT + 1 + NSLOT]
        ssem = rest[NSLOT + 1 + NSLOT:]
        cid = lax.axis_index("c")
        sid = lax.axis_index("s")
        base = sid * rpt
        my_xs = xs_hbm.at[t, cid]

        pltpu.sync_copy(src_hbm.at[sid], src_v)
        pltpu.sync_copy(dst_hbm.at[sid], dst_v)

        def gath(k, slot):
            pltpu.async_copy(my_xs.at[src_v.at[k]], bufs[slot], gsem[slot])

        def gwait(k, slot):
            pltpu.make_async_copy(
                my_xs.at[src_v.at[k]], bufs[slot], gsem[slot]).wait()

        def scat(k, slot):
            pltpu.async_copy(bufs[slot], agg_sh.at[dst_v.at[k]], ssem[slot],
                             add=True)

        def swait(k, slot):
            pltpu.make_async_copy(
                bufs[slot], agg_sh.at[dst_v.at[k]], ssem[slot]).wait()

        for k in range(LOOK):
            gath(k, k)

        zbuf = bufs[NSLOT - 1]

        def zfill(j, carry):
            r = j // (FH // 16)
            col = j % (FH // 16)
            zbuf[r, pl.ds(col * 16, 16)] = jnp.zeros((16,), jnp.float32)
            return carry
        lax.fori_loop(0, CHUNK * (FH // 16), zfill, 0)

        def zcopy(k, carry):
            pltpu.sync_copy(zbuf, agg_sh.at[pl.ds(base + k * CHUNK, CHUNK)])
            return carry
        lax.fori_loop(0, nstrip, zcopy, 0)
        plsc.subcore_barrier()

        for k in range(LOOK):
            gwait(k, k)
            scat(k, k)
            gath(k + LOOK, k + LOOK)

        def step(k, slot):
            gwait(k, slot)
            scat(k, slot)
            old = (slot + LOOK) % NSLOT
            swait(k - LOOK, old)
            gath(k + LOOK, old)

        def ring(i, carry):
            kb = LOOK + i * NSLOT
            for j in range(NSLOT):
                step(kb + j, (LOOK + j) % NSLOT)
            return carry
        lax.fori_loop(0, (nch - 2 * LOOK) // NSLOT, ring, 0)

        for k in range(nch - LOOK, nch):
            slot = k % NSLOT
            gwait(k, slot)
            scat(k, slot)
            swait(k - LOOK, (slot + LOOK) % NSLOT)
        for k in range(nch - LOOK, nch):
            swait(k, k % NSLOT)
        plsc.subcore_barrier()

        def out_copy(k, carry):
            sl = pl.ds(base + k * CHUNK, CHUNK)
            pltpu.sync_copy(agg_sh.at[sl], bufs[0])
            pltpu.sync_copy(bufs[0], out_hbm.at[cid, sl])
            return carry
        lax.fori_loop(0, nstrip, out_copy, 0)

    return scat_kernel(xs_all, src_r, dst_r)


def _proj_first_body(f_ref, w_ref, b_ref, d0_ref, d1_ref, xs_ref, norm_ref):
    deg = d0_ref[0][:, 0:1] + d1_ref[0][:, 0:1]
    nrm = lax.rsqrt(jnp.clip(deg, 1.0, None))
    x = jnp.dot(f_ref[0], w_ref[...], preferred_element_type=jnp.float32)
    xs = (x + b_ref[...]) * nrm
    xs_ref[0, 0] = xs[:, 0:FH]
    xs_ref[0, 1] = xs[:, FH:2 * FH]
    norm_ref[...] = nrm


def _tc_project_first(feat, w, b2d, degp, *, br):
    t, n, din = feat.shape
    hid = w.shape[1]
    nblk = n // br
    return pl.pallas_call(
        _proj_first_body,
        grid=(t, nblk),
        in_specs=[
            pl.BlockSpec((1, br, din), lambda i, r: (i, r, 0)),
            pl.BlockSpec((din, hid), lambda i, r: (0, 0)),
            pl.BlockSpec((1, hid), lambda i, r: (0, 0)),
            pl.BlockSpec((1, br, 16), lambda i, r: (0, r, 0)),
            pl.BlockSpec((1, br, 16), lambda i, r: (1, r, 0)),
        ],
        out_specs=[
            pl.BlockSpec((1, 2, br, FH), lambda i, r: (i, 0, r, 0)),
            pl.BlockSpec((br, 1), lambda i, r: (r, 0)),
        ],
        out_shape=[
            jax.ShapeDtypeStruct((t, 2, n, FH), jnp.float32),
            jax.ShapeDtypeStruct((n, 1), jnp.float32),
        ],
    )(feat, w, b2d, degp, degp)


def _proj_rest_body(f_ref, w_ref, b_ref, nrm_ref, xs_ref):
    x = jnp.dot(f_ref[0], w_ref[...], preferred_element_type=jnp.float32)
    xs = (x + b_ref[...]) * nrm_ref[...]
    xs_ref[0, 0] = xs[:, 0:FH]
    xs_ref[0, 1] = xs[:, FH:2 * FH]


def _tc_project_rest(feat, w, b2d, nrm, *, br):
    t, n, din = feat.shape
    hid = w.shape[1]
    nblk = n // br
    return pl.pallas_call(
        _proj_rest_body,
        grid=(t, nblk),
        in_specs=[
            pl.BlockSpec((1, br, din), lambda i, r: (i, r, 0)),
            pl.BlockSpec((din, hid), lambda i, r: (0, 0)),
            pl.BlockSpec((1, hid), lambda i, r: (0, 0)),
            pl.BlockSpec((br, 1), lambda i, r: (r, 0)),
        ],
        out_specs=pl.BlockSpec((1, 2, br, FH), lambda i, r: (i, 0, r, 0)),
        out_shape=jax.ShapeDtypeStruct((t, 2, n, FH), jnp.float32),
    )(feat, w, b2d, nrm)


def _lstm_body(p0_ref, p1_ref, nrm_ref, h_ref, c_ref,
               wi_ref, wh_ref, bs_ref, h2_ref, c2_ref):
    hid = h_ref.shape[-1]
    agg = jnp.concatenate([p0_ref[0], p1_ref[0]], axis=1)
    g = jnp.maximum(agg * nrm_ref[...], 0.0)
    gates = (jnp.dot(g, wi_ref[...], preferred_element_type=jnp.float32)
             + jnp.dot(h_ref[...], wh_ref[...],
                       preferred_element_type=jnp.float32)
             + bs_ref[...])
    i_g = jax.nn.sigmoid(gates[:, 0:hid])
    f_g = jax.nn.sigmoid(gates[:, hid:2 * hid])
    g_g = jnp.tanh(gates[:, 2 * hid:3 * hid])
    o_g = jax.nn.sigmoid(gates[:, 3 * hid:4 * hid])
    cn = f_g * c_ref[...] + i_g * g_g
    h2_ref[...] = o_g * jnp.tanh(cn)
    c2_ref[...] = cn


def _tc_lstm(part, nrm, h, c, wi_t, wh_t, bsum, *, br):
    n, hid = h.shape
    nblk = n // br
    return pl.pallas_call(
        _lstm_body,
        grid=(nblk,),
        in_specs=[
            pl.BlockSpec((1, br, FH), lambda r: (0, r, 0)),
            pl.BlockSpec((1, br, FH), lambda r: (1, r, 0)),
            pl.BlockSpec((br, 1), lambda r: (r, 0)),
            pl.BlockSpec((br, hid), lambda r: (r, 0)),
            pl.BlockSpec((br, hid), lambda r: (r, 0)),
            pl.BlockSpec((hid, 4 * hid), lambda r: (0, 0)),
            pl.BlockSpec((hid, 4 * hid), lambda r: (0, 0)),
            pl.BlockSpec((1, 4 * hid), lambda r: (0, 0)),
        ],
        out_specs=[
            pl.BlockSpec((br, hid), lambda r: (r, 0)),
            pl.BlockSpec((br, hid), lambda r: (r, 0)),
        ],
        out_shape=[
            jax.ShapeDtypeStruct((n, hid), jnp.float32),
            jax.ShapeDtypeStruct((n, hid), jnp.float32),
        ],
    )(part, part, nrm, h, c, wi_t, wh_t, bsum)


def _mlp_body(h_ref, w1_ref, b1_ref, w2_ref, b2_ref, o_ref):
    z = jnp.maximum(
        jnp.dot(h_ref[...], w1_ref[...], preferred_element_type=jnp.float32)
        + b1_ref[...], 0.0)
    o_ref[...] = (jnp.dot(z, w2_ref[...], preferred_element_type=jnp.float32)
                  + b2_ref[...])


def _tc_mlp(h, w1, b1_2d, w2, b2_2d, *, br):
    n, hid = h.shape
    cls = w1.shape[1]
    dout = w2.shape[1]
    nblk = n // br
    return pl.pallas_call(
        _mlp_body,
        grid=(nblk,),
        in_specs=[
            pl.BlockSpec((br, hid), lambda r: (r, 0)),
            pl.BlockSpec((hid, cls), lambda r: (0, 0)),
            pl.BlockSpec((1, cls), lambda r: (0, 0)),
            pl.BlockSpec((cls, dout), lambda r: (0, 0)),
            pl.BlockSpec((1, dout), lambda r: (0, 0)),
        ],
        out_specs=pl.BlockSpec((br, dout), lambda r: (r, 0)),
        out_shape=jax.ShapeDtypeStruct((n, dout), jnp.float32),
    )(h, w1, b1_2d, w2, b2_2d)


def kernel(feat_list, edge_index, n_step, W_gcn, b_gcn, Wi, Wh, bi, bh,
           W1, b1, W2, b2):
    del n_step
    t_steps, n, _ = feat_list.shape
    hid = W_gcn.shape[1]
    br = 1000 if n % 1000 == 0 else 8 * (n // 8)

    src = edge_index[0]
    dst = edge_index[1]
    e = src.shape[0]

    npad = _round_up(n + 1, NS * CHUNK)
    epad = _round_up(e, NS * CHUNK * NSLOT * NC)
    nch = epad // (NS * CHUNK)
    pad = epad - e
    pad_i = jnp.arange(pad, dtype=jnp.int32)
    src_p = jnp.concatenate([src, pad_i % n])
    dst_p = jnp.concatenate([dst, n + pad_i % (npad - n)])
    src_r = src_p.reshape(NS, nch, CHUNK)
    dst_r = dst_p.reshape(NS, nch, CHUNK)

    degp = _sc_degree(dst_r, npad=npad, nch=nch)

    b2d = b_gcn.reshape(1, hid)
    xs_t0, nrm = _tc_project_first(feat_list[0:1], W_gcn, b2d, degp, br=br)
    xs_rest = _tc_project_rest(feat_list[1:], W_gcn, b2d, nrm, br=br)

    wi_t = Wi.T
    wh_t = Wh.T
    bsum = (bi + bh).reshape(1, 4 * hid)
    h = jnp.zeros((n, hid), jnp.float32)
    c = jnp.zeros((n, hid), jnp.float32)
    for t in range(t_steps):
        xs_arr, tt = (xs_t0, 0) if t == 0 else (xs_rest, t - 1)
        part = _sc_scatter(xs_arr, src_r, dst_r, t=tt, npad=npad, nch=nch)
        h, c = _tc_lstm(part, nrm, h, c, wi_t, wh_t, bsum, br=br)

    return _tc_mlp(h, W1, b1.reshape(1, -1), W2, b2.reshape(1, -1), br=br)


# speedup vs baseline: 12.3925x; 1.0738x over previous
import functools

import jax
import jax.numpy as jnp
from jax import lax
from jax.experimental import pallas as pl
from jax.experimental.pallas import tpu as pltpu
from jax.experimental.pallas import tpu_sc as plsc

NC = 2
NS = 16
CHUNK = 128
FH = 64
NSLOT = 4
LOOK = 2


def _round_up(x, m):
    return (x + m - 1) // m * m


def _sc_degree(dst_r, *, npad, nch):
    mesh = plsc.VectorSubcoreMesh(core_axis_name="c", subcore_axis_name="s")
    rpt = npad // NS
    nstrip = rpt // CHUNK
    nchc = nch // NC

    @functools.partial(
        pl.kernel,
        out_type=jax.ShapeDtypeStruct((NC, npad, 16), jnp.float32),
        mesh=mesh,
        scratch_types=[
            pltpu.VMEM((nchc, CHUNK), jnp.int32),
            pltpu.VMEM((CHUNK, 16), jnp.float32),
            pltpu.VMEM((CHUNK, 16), jnp.float32),
            pltpu.VMEM_SHARED((npad, 16), jnp.float32),
            pltpu.SemaphoreType.DMA,
        ],
        compiler_params=pltpu.CompilerParams(use_tc_tiling_on_sc=False),
    )
    def deg_kernel(dst_hbm, out_hbm, dst_v, ones_v, zero_v, deg_sh, sem):
        cid = lax.axis_index("c")
        sid = lax.axis_index("s")
        base = sid * rpt
        pltpu.sync_copy(dst_hbm.at[sid, pl.ds(cid * nchc, nchc)], dst_v)

        def fill(buf, val):
            def body(j, carry):
                buf[j] = jnp.full((16,), val, jnp.float32)
                return carry
            lax.fori_loop(0, CHUNK, body, 0)

        fill(zero_v, 0.0)
        fill(ones_v, 1.0)

        def zcopy(k, carry):
            pltpu.sync_copy(zero_v, deg_sh.at[pl.ds(base + k * CHUNK, CHUNK)])
            return carry
        lax.fori_loop(0, nstrip, zcopy, 0)
        plsc.subcore_barrier()

        def round8(r, carry):
            for j in range(8):
                pltpu.async_copy(ones_v, deg_sh.at[dst_v.at[r * 8 + j]], sem,
                                 add=True)
            for j in range(8):
                pltpu.make_async_copy(
                    ones_v, deg_sh.at[dst_v.at[r * 8 + j]], sem).wait()
            return carry
        lax.fori_loop(0, nchc // 8, round8, 0)
        plsc.subcore_barrier()

        def out_copy(k, carry):
            sl = pl.ds(base + k * CHUNK, CHUNK)
            pltpu.sync_copy(deg_sh.at[sl], ones_v)
            pltpu.sync_copy(ones_v, out_hbm.at[cid, sl])
            return carry
        lax.fori_loop(0, nstrip, out_copy, 0)

    return deg_kernel(dst_r)


def _sc_scatter(xs_all, src_r, dst_r, *, t, npad, nch):
    mesh = plsc.VectorSubcoreMesh(core_axis_name="c", subcore_axis_name="s")
    rpt = npad // NS
    nstrip = rpt // CHUNK

    @functools.partial(
        pl.kernel,
        out_type=jax.ShapeDtypeStruct((NC, npad, FH), jnp.float32),
        mesh=mesh,
        scratch_types=[
            pltpu.VMEM((nch, CHUNK), jnp.int32),
            pltpu.VMEM((nch, CHUNK), jnp.int32),
        ] + [pltpu.VMEM((CHUNK, FH), jnp.float32) for _ in range(NSLOT)] + [
            pltpu.VMEM_SHARED((npad, FH), jnp.float32),
        ] + [pltpu.SemaphoreType.DMA for _ in range(2 * NSLOT)],
        compiler_params=pltpu.CompilerParams(use_tc_tiling_on_sc=False),
    )
    def scat_kernel(xs_hbm, src_hbm, dst_hbm, out_hbm,
                    src_v, dst_v, *rest):
        bufs = rest[:NSLOT]
        agg_sh = rest[NSLOT]
        gsem = rest[NSLOT + 1:NSLOT + 1 + NSLOT]
        ssem = rest[NSLOT + 1 + NSLOT:]
        cid = lax.axis_index("c")
        sid = lax.axis_index("s")
        base = sid * rpt
        my_xs = xs_hbm.at[t, cid]

        pltpu.sync_copy(src_hbm.at[sid], src_v)
        pltpu.sync_copy(dst_hbm.at[sid], dst_v)

        def gath(k, slot):
            pltpu.async_copy(my_xs.at[src_v.at[k]], bufs[slot], gsem[slot])

        def gwait(k, slot):
            pltpu.make_async_copy(
                my_xs.at[src_v.at[k]], bufs[slot], gsem[slot]).wait()

        def scat(k, slot):
            pltpu.async_copy(bufs[slot], agg_sh.at[dst_v.at[k]], ssem[slot],
                             add=True)

        def swait(k, slot):
            pltpu.make_async_copy(
                bufs[slot], agg_sh.at[dst_v.at[k]], ssem[slot]).wait()

        for k in range(LOOK):
            gath(k, k)

        zbuf = bufs[NSLOT - 1]

        def zfill(j, carry):
            r = j // (FH // 16)
            col = j % (FH // 16)
            zbuf[r, pl.ds(col * 16, 16)] = jnp.zeros((16,), jnp.float32)
            return carry
        lax.fori_loop(0, CHUNK * (FH // 16), zfill, 0)

        def zcopy(k, carry):
            pltpu.sync_copy(zbuf, agg_sh.at[pl.ds(base + k * CHUNK, CHUNK)])
            return carry
        lax.fori_loop(0, nstrip, zcopy, 0)
        plsc.subcore_barrier()

        for k in range(LOOK):
            gwait(k, k)
            scat(k, k)
            gath(k + LOOK, k + LOOK)

        def step(k, slot):
            gwait(k, slot)
            scat(k, slot)
            old = (slot + LOOK) % NSLOT
            swait(k - LOOK, old)
            gath(k + LOOK, old)

        def ring(i, carry):
            kb = LOOK + i * NSLOT
            for j in range(NSLOT):
                step(kb + j, (LOOK + j) % NSLOT)
            return carry
        lax.fori_loop(0, (nch - 2 * LOOK) // NSLOT, ring, 0)

        for k in range(nch - LOOK, nch):
            slot = k % NSLOT
            gwait(k, slot)
            scat(k, slot)
            swait(k - LOOK, (slot + LOOK) % NSLOT)
        for k in range(nch - LOOK, nch):
            swait(k, k % NSLOT)
        plsc.subcore_barrier()

        def out_copy(k, carry):
            sl = pl.ds(base + k * CHUNK, CHUNK)
            pltpu.sync_copy(agg_sh.at[sl], bufs[0])
            pltpu.sync_copy(bufs[0], out_hbm.at[cid, sl])
            return carry
        lax.fori_loop(0, nstrip, out_copy, 0)

    return scat_kernel(xs_all, src_r, dst_r)


def _proj_first_body(f_ref, w_ref, b_ref, d0_ref, d1_ref, xs_ref, norm_ref):
    deg = d0_ref[0][:, 0:1] + d1_ref[0][:, 0:1]
    nrm = lax.rsqrt(jnp.clip(deg, 1.0, None))
    x = jnp.dot(f_ref[0], w_ref[...], preferred_element_type=jnp.float32)
    xs = (x + b_ref[...]) * nrm
    xs_ref[0, 0] = xs[:, 0:FH]
    xs_ref[0, 1] = xs[:, FH:2 * FH]
    norm_ref[...] = nrm


def _tc_project_first(feat, w, b2d, degp, *, br):
    t, n, din = feat.shape
    hid = w.shape[1]
    nblk = n // br
    return pl.pallas_call(
        _proj_first_body,
        grid=(t, nblk),
        in_specs=[
            pl.BlockSpec((1, br, din), lambda i, r: (i, r, 0)),
            pl.BlockSpec((din, hid), lambda i, r: (0, 0)),
            pl.BlockSpec((1, hid), lambda i, r: (0, 0)),
            pl.BlockSpec((1, br, 16), lambda i, r: (0, r, 0)),
            pl.BlockSpec((1, br, 16), lambda i, r: (1, r, 0)),
        ],
        out_specs=[
            pl.BlockSpec((1, 2, br, FH), lambda i, r: (i, 0, r, 0)),
            pl.BlockSpec((br, 1), lambda i, r: (r, 0)),
        ],
        out_shape=[
            jax.ShapeDtypeStruct((t, 2, n, FH), jnp.float32),
            jax.ShapeDtypeStruct((n, 1), jnp.float32),
        ],
    )(feat, w, b2d, degp, degp)


def _proj_rest_body(f_ref, w_ref, b_ref, nrm_ref, xs_ref):
    x = jnp.dot(f_ref[0], w_ref[...], preferred_element_type=jnp.float32)
    xs = (x + b_ref[...]) * nrm_ref[...]
    xs_ref[0, 0] = xs[:, 0:FH]
    xs_ref[0, 1] = xs[:, FH:2 * FH]


def _tc_project_rest(feat, w, b2d, nrm, *, br):
    t, n, din = feat.shape
    hid = w.shape[1]
    nblk = n // br
    return pl.pallas_call(
        _proj_rest_body,
        grid=(t, nblk),
        in_specs=[
            pl.BlockSpec((1, br, din), lambda i, r: (i, r, 0)),
            pl.BlockSpec((din, hid), lambda i, r: (0, 0)),
            pl.BlockSpec((1, hid), lambda i, r: (0, 0)),
            pl.BlockSpec((br, 1), lambda i, r: (r, 0)),
        ],
        out_specs=pl.BlockSpec((1, 2, br, FH), lambda i, r: (i, 0, r, 0)),
        out_shape=jax.ShapeDtypeStruct((t, 2, n, FH), jnp.float32),
    )(feat, w, b2d, nrm)


def _lstm_head_body(nrm_ref, wi_ref, wh_ref, bs_ref, w1_ref, b1_ref,
                    w2_ref, b2_ref, *rest):
    part_refs = rest[:-1]
    o_ref = rest[-1]
    hid = wh_ref.shape[0]
    nrm = nrm_ref[...]
    h = None
    c = None
    for p_ref in part_refs:
        agg = jnp.concatenate([p_ref[0], p_ref[1]], axis=1)
        g = jnp.maximum(agg * nrm, 0.0)
        gates = jnp.dot(g, wi_ref[...], preferred_element_type=jnp.float32)
        if h is not None:
            gates = gates + jnp.dot(h, wh_ref[...],
                                    preferred_element_type=jnp.float32)
        gates = gates + bs_ref[...]
        i_g = jax.nn.sigmoid(gates[:, 0:hid])
        f_g = jax.nn.sigmoid(gates[:, hid:2 * hid])
        g_g = jnp.tanh(gates[:, 2 * hid:3 * hid])
        o_g = jax.nn.sigmoid(gates[:, 3 * hid:4 * hid])
        c = i_g * g_g if c is None else f_g * c + i_g * g_g
        h = o_g * jnp.tanh(c)
    z = jnp.maximum(
        jnp.dot(h, w1_ref[...], preferred_element_type=jnp.float32)
        + b1_ref[...], 0.0)
    o_ref[...] = (jnp.dot(z, w2_ref[...], preferred_element_type=jnp.float32)
                  + b2_ref[...])


def _tc_lstm_head(parts, nrm, wi_t, wh_t, bsum, w1, b1_2d, w2, b2_2d, *, br):
    n = nrm.shape[0]
    hid = wh_t.shape[0]
    cls = w1.shape[1]
    dout = w2.shape[1]
    nblk = n // br
    return pl.pallas_call(
        _lstm_head_body,
        grid=(nblk,),
        in_specs=[
            pl.BlockSpec((br, 1), lambda r: (r, 0)),
            pl.BlockSpec((hid, 4 * hid), lambda r: (0, 0)),
            pl.BlockSpec((hid, 4 * hid), lambda r: (0, 0)),
            pl.BlockSpec((1, 4 * hid), lambda r: (0, 0)),
            pl.BlockSpec((hid, cls), lambda r: (0, 0)),
            pl.BlockSpec((1, cls), lambda r: (0, 0)),
            pl.BlockSpec((cls, dout), lambda r: (0, 0)),
            pl.BlockSpec((1, dout), lambda r: (0, 0)),
        ] + [pl.BlockSpec((2, br, FH), lambda r: (0, r, 0))
             for _ in parts],
        out_specs=pl.BlockSpec((br, dout), lambda r: (r, 0)),
        out_shape=jax.ShapeDtypeStruct((n, dout), jnp.float32),
    )(nrm, wi_t, wh_t, bsum, w1, b1_2d, w2, b2_2d, *parts)


def kernel(feat_list, edge_index, n_step, W_gcn, b_gcn, Wi, Wh, bi, bh,
           W1, b1, W2, b2):
    del n_step
    t_steps, n, _ = feat_list.shape
    hid = W_gcn.shape[1]
    br = 1000 if n % 1000 == 0 else 8 * (n // 8)

    src = edge_index[0]
    dst = edge_index[1]
    e = src.shape[0]

    npad = _round_up(n + 1, NS * CHUNK)
    epad = _round_up(e, NS * CHUNK * NSLOT * NC)
    nch = epad // (NS * CHUNK)
    pad = epad - e
    dmask = 1
    while dmask * 2 <= npad - n:
        dmask *= 2
    smask = 1
    while smask * 2 <= n:
        smask *= 2
    pad_i = jnp.arange(pad, dtype=jnp.int32)
    src_p = jnp.concatenate([src, pad_i & (smask - 1)])
    dst_p = jnp.concatenate([dst, n + (pad_i & (dmask - 1))])
    src_r = src_p.reshape(NS, nch, CHUNK)
    dst_r = dst_p.reshape(NS, nch, CHUNK)

    degp = _sc_degree(dst_r, npad=npad, nch=nch)

    b2d = b_gcn.reshape(1, hid)
    xs_t0, nrm = _tc_project_first(feat_list[0:1], W_gcn, b2d, degp, br=br)
    parts = [_sc_scatter(xs_t0, src_r, dst_r, t=0, npad=npad, nch=nch)]
    xs_rest = _tc_project_rest(feat_list[1:], W_gcn, b2d, nrm, br=br)
    for t in range(1, t_steps):
        parts.append(
            _sc_scatter(xs_rest, src_r, dst_r, t=t - 1, npad=npad, nch=nch))

    return _tc_lstm_head(parts, nrm, Wi.T, Wh.T,
                         (bi + bh).reshape(1, 4 * hid),
                         W1, b1.reshape(1, -1), W2, b2.reshape(1, -1), br=br)


# speedup vs baseline: 17.5440x; 1.4157x over previous
import functools

import jax
import jax.numpy as jnp
from jax import lax
from jax.experimental import pallas as pl
from jax.experimental.pallas import tpu as pltpu
from jax.experimental.pallas import tpu_sc as plsc

NC = 2
NS = 16
CHUNK = 128
FH = 64
NSLOT = 4
LOOK = 2


def _round_up(x, m):
    return (x + m - 1) // m * m


def _sc_degree(dst_r, *, npad, nch):
    mesh = plsc.VectorSubcoreMesh(core_axis_name="c", subcore_axis_name="s")
    rpt = npad // NS
    nstrip = rpt // CHUNK
    nchc = nch // NC

    @functools.partial(
        pl.kernel,
        out_type=jax.ShapeDtypeStruct((NC, npad, 16), jnp.float32),
        mesh=mesh,
        scratch_types=[
            pltpu.VMEM((nchc, CHUNK), jnp.int32),
            pltpu.VMEM((CHUNK, 16), jnp.float32),
            pltpu.VMEM((CHUNK, 16), jnp.float32),
            pltpu.VMEM_SHARED((npad, 16), jnp.float32),
            pltpu.SemaphoreType.DMA,
        ],
        compiler_params=pltpu.CompilerParams(use_tc_tiling_on_sc=False),
    )
    def deg_kernel(dst_hbm, out_hbm, dst_v, ones_v, zero_v, deg_sh, sem):
        cid = lax.axis_index("c")
        sid = lax.axis_index("s")
        base = sid * rpt
        pltpu.sync_copy(dst_hbm.at[sid, pl.ds(cid * nchc, nchc)], dst_v)

        def fill(buf, val):
            def body(j, carry):
                buf[j] = jnp.full((16,), val, jnp.float32)
                return carry
            lax.fori_loop(0, CHUNK, body, 0)

        fill(zero_v, 0.0)
        fill(ones_v, 1.0)

        def zcopy(k, carry):
            pltpu.sync_copy(zero_v, deg_sh.at[pl.ds(base + k * CHUNK, CHUNK)])
            return carry
        lax.fori_loop(0, nstrip, zcopy, 0)
        plsc.subcore_barrier()

        def round8(r, carry):
            for j in range(8):
                pltpu.async_copy(ones_v, deg_sh.at[dst_v.at[r * 8 + j]], sem,
                                 add=True)
            for j in range(8):
                pltpu.make_async_copy(
                    ones_v, deg_sh.at[dst_v.at[r * 8 + j]], sem).wait()
            return carry
        lax.fori_loop(0, nchc // 8, round8, 0)
        plsc.subcore_barrier()

        def out_copy(k, carry):
            sl = pl.ds(base + k * CHUNK, CHUNK)
            pltpu.sync_copy(deg_sh.at[sl], ones_v)
            pltpu.sync_copy(ones_v, out_hbm.at[cid, sl])
            return carry
        lax.fori_loop(0, nstrip, out_copy, 0)

    return deg_kernel(dst_r)


def _sc_scatter(xs_all, src_r, dst_r, *, t, npad, nch):
    mesh = plsc.VectorSubcoreMesh(core_axis_name="c", subcore_axis_name="s")
    rpt = npad // NS
    nstrip = rpt // CHUNK
    hid = xs_all.shape[-1]
    nchc = nch // NC

    @functools.partial(
        pl.kernel,
        out_type=jax.ShapeDtypeStruct((NC, npad, hid), jnp.bfloat16),
        mesh=mesh,
        scratch_types=[
            pltpu.VMEM((nchc, CHUNK), jnp.int32),
            pltpu.VMEM((nchc, CHUNK), jnp.int32),
        ] + [pltpu.VMEM((CHUNK, hid), jnp.bfloat16) for _ in range(NSLOT)] + [
            pltpu.VMEM_SHARED((npad, hid), jnp.bfloat16),
        ] + [pltpu.SemaphoreType.DMA for _ in range(2 * NSLOT)],
        compiler_params=pltpu.CompilerParams(use_tc_tiling_on_sc=False),
    )
    def scat_kernel(xs_hbm, src_hbm, dst_hbm, out_hbm,
                    src_v, dst_v, *rest):
        bufs = rest[:NSLOT]
        agg_sh = rest[NSLOT]
        gsem = rest[NSLOT + 1:NSLOT + 1 + NSLOT]
        ssem = rest[NSLOT + 1 + NSLOT:]
        cid = lax.axis_index("c")
        sid = lax.axis_index("s")
        base = sid * rpt
        my_xs = xs_hbm.at[t]

        pltpu.sync_copy(src_hbm.at[sid, pl.ds(cid * nchc, nchc)], src_v)
        pltpu.sync_copy(dst_hbm.at[sid, pl.ds(cid * nchc, nchc)], dst_v)

        def gath(k, slot):
            pltpu.async_copy(my_xs.at[src_v.at[k]], bufs[slot], gsem[slot])

        def gwait(k, slot):
            pltpu.make_async_copy(
                my_xs.at[src_v.at[k]], bufs[slot], gsem[slot]).wait()

        def scat(k, slot):
            pltpu.async_copy(bufs[slot], agg_sh.at[dst_v.at[k]], ssem[slot],
                             add=True)

        def swait(k, slot):
            pltpu.make_async_copy(
                bufs[slot], agg_sh.at[dst_v.at[k]], ssem[slot]).wait()

        for k in range(LOOK):
            gath(k, k)

        zbuf = bufs[NSLOT - 1]

        def zfill(j, carry):
            r = j // (hid // 32)
            col = j % (hid // 32)
            zbuf[r, pl.ds(col * 32, 32)] = jnp.zeros((32,), jnp.bfloat16)
            return carry
        lax.fori_loop(0, CHUNK * (hid // 32), zfill, 0)

        def zcopy(k, carry):
            pltpu.sync_copy(zbuf, agg_sh.at[pl.ds(base + k * CHUNK, CHUNK)])
            return carry
        lax.fori_loop(0, nstrip, zcopy, 0)
        plsc.subcore_barrier()

        for k in range(LOOK):
            gwait(k, k)
            scat(k, k)
            gath(k + LOOK, k + LOOK)

        def step(k, slot):
            gwait(k, slot)
            scat(k, slot)
            old = (slot + LOOK) % NSLOT
            swait(k - LOOK, old)
            gath(k + LOOK, old)

        def ring(i, carry):
            kb = LOOK + i * NSLOT
            for j in range(NSLOT):
                step(kb + j, (LOOK + j) % NSLOT)
            return carry
        lax.fori_loop(0, (nchc - 2 * LOOK) // NSLOT, ring, 0)

        for k in range(nchc - LOOK, nchc):
            slot = k % NSLOT
            gwait(k, slot)
            scat(k, slot)
            swait(k - LOOK, (slot + LOOK) % NSLOT)
        for k in range(nchc - LOOK, nchc):
            swait(k, k % NSLOT)
        plsc.subcore_barrier()

        def out_copy(k, carry):
            sl = pl.ds(base + k * CHUNK, CHUNK)
            pltpu.sync_copy(agg_sh.at[sl], bufs[0])
            pltpu.sync_copy(bufs[0], out_hbm.at[cid, sl])
            return carry
        lax.fori_loop(0, nstrip, out_copy, 0)

    return scat_kernel(xs_all, src_r, dst_r)


def _proj_first_body(f_ref, w_ref, b_ref, d0_ref, d1_ref, xs_ref, norm_ref):
    deg = d0_ref[0][:, 0:1] + d1_ref[0][:, 0:1]
    nrm = lax.rsqrt(jnp.clip(deg, 1.0, None))
    x = jnp.dot(f_ref[0], w_ref[...], preferred_element_type=jnp.float32)
    xs_ref[0] = ((x + b_ref[...]) * nrm).astype(jnp.bfloat16)
    norm_ref[...] = nrm


def _tc_project_first(feat, w, b2d, degp, *, br):
    t, n, din = feat.shape
    hid = w.shape[1]
    nblk = n // br
    return pl.pallas_call(
        _proj_first_body,
        grid=(t, nblk),
        in_specs=[
            pl.BlockSpec((1, br, din), lambda i, r: (i, r, 0)),
            pl.BlockSpec((din, hid), lambda i, r: (0, 0)),
            pl.BlockSpec((1, hid), lambda i, r: (0, 0)),
            pl.BlockSpec((1, br, 16), lambda i, r: (0, r, 0)),
            pl.BlockSpec((1, br, 16), lambda i, r: (1, r, 0)),
        ],
        out_specs=[
            pl.BlockSpec((1, br, hid), lambda i, r: (i, r, 0)),
            pl.BlockSpec((br, 1), lambda i, r: (r, 0)),
        ],
        out_shape=[
            jax.ShapeDtypeStruct((t, n, hid), jnp.bfloat16),
            jax.ShapeDtypeStruct((n, 1), jnp.float32),
        ],
    )(feat, w, b2d, degp, degp)


def _proj_rest_body(f_ref, w_ref, b_ref, nrm_ref, xs_ref):
    x = jnp.dot(f_ref[0], w_ref[...], preferred_element_type=jnp.float32)
    xs_ref[0] = ((x + b_ref[...]) * nrm_ref[...]).astype(jnp.bfloat16)


def _tc_project_rest(feat, w, b2d, nrm, *, br):
    t, n, din = feat.shape
    hid = w.shape[1]
    nblk = n // br
    return pl.pallas_call(
        _proj_rest_body,
        grid=(t, nblk),
        in_specs=[
            pl.BlockSpec((1, br, din), lambda i, r: (i, r, 0)),
            pl.BlockSpec((din, hid), lambda i, r: (0, 0)),
            pl.BlockSpec((1, hid), lambda i, r: (0, 0)),
            pl.BlockSpec((br, 1), lambda i, r: (r, 0)),
        ],
        out_specs=pl.BlockSpec((1, br, hid), lambda i, r: (i, r, 0)),
        out_shape=jax.ShapeDtypeStruct((t, n, hid), jnp.bfloat16),
    )(feat, w, b2d, nrm)


def _lstm_head_body(nrm_ref, wi_ref, wh_ref, bs_ref, w1_ref, b1_ref,
                    w2_ref, b2_ref, *rest):
    part_refs = rest[:-1]
    o_ref = rest[-1]
    hid = wh_ref.shape[0]
    nrm = nrm_ref[...]
    h = None
    c = None
    for p_ref in part_refs:
        agg = (p_ref[0].astype(jnp.float32)
               + p_ref[1].astype(jnp.float32))
        g = jnp.maximum(agg * nrm, 0.0)
        gates = jnp.dot(g, wi_ref[...], preferred_element_type=jnp.float32)
        if h is not None:
            gates = gates + jnp.dot(h, wh_ref[...],
                                    preferred_element_type=jnp.float32)
        gates = gates + bs_ref[...]
        i_g = jax.nn.sigmoid(gates[:, 0:hid])
        f_g = jax.nn.sigmoid(gates[:, hid:2 * hid])
        g_g = jnp.tanh(gates[:, 2 * hid:3 * hid])
        o_g = jax.nn.sigmoid(gates[:, 3 * hid:4 * hid])
        c = i_g * g_g if c is None else f_g * c + i_g * g_g
        h = o_g * jnp.tanh(c)
    z = jnp.maximum(
        jnp.dot(h, w1_ref[...], preferred_element_type=jnp.float32)
        + b1_ref[...], 0.0)
    o_ref[...] = (jnp.dot(z, w2_ref[...], preferred_element_type=jnp.float32)
                  + b2_ref[...])


def _tc_lstm_head(parts, nrm, wi_t, wh_t, bsum, w1, b1_2d, w2, b2_2d, *, br):
    n = nrm.shape[0]
    hid = wh_t.shape[0]
    cls = w1.shape[1]
    dout = w2.shape[1]
    nblk = n // br
    return pl.pallas_call(
        _lstm_head_body,
        grid=(nblk,),
        in_specs=[
            pl.BlockSpec((br, 1), lambda r: (r, 0)),
            pl.BlockSpec((hid, 4 * hid), lambda r: (0, 0)),
            pl.BlockSpec((hid, 4 * hid), lambda r: (0, 0)),
            pl.BlockSpec((1, 4 * hid), lambda r: (0, 0)),
            pl.BlockSpec((hid, cls), lambda r: (0, 0)),
            pl.BlockSpec((1, cls), lambda r: (0, 0)),
            pl.BlockSpec((cls, dout), lambda r: (0, 0)),
            pl.BlockSpec((1, dout), lambda r: (0, 0)),
        ] + [pl.BlockSpec((2, br, hid), lambda r: (0, r, 0))
             for _ in parts],
        out_specs=pl.BlockSpec((br, dout), lambda r: (r, 0)),
        out_shape=jax.ShapeDtypeStruct((n, dout), jnp.float32),
    )(nrm, wi_t, wh_t, bsum, w1, b1_2d, w2, b2_2d, *parts)


def kernel(feat_list, edge_index, n_step, W_gcn, b_gcn, Wi, Wh, bi, bh,
           W1, b1, W2, b2):
    del n_step
    t_steps, n, _ = feat_list.shape
    hid = W_gcn.shape[1]
    br = 1000 if n % 1000 == 0 else 8 * (n // 8)

    src = edge_index[0]
    dst = edge_index[1]
    e = src.shape[0]

    npad = _round_up(n + 1, NS * CHUNK)
    epad = _round_up(e, NS * CHUNK * NSLOT * NC)
    nch = epad // (NS * CHUNK)
    pad = epad - e
    dmask = 1
    while dmask * 2 <= npad - n:
        dmask *= 2
    smask = 1
    while smask * 2 <= n:
        smask *= 2
    pad_i = jnp.arange(pad, dtype=jnp.int32)
    src_p = jnp.concatenate([src, pad_i & (smask - 1)])
    dst_p = jnp.concatenate([dst, n + (pad_i & (dmask - 1))])
    src_r = src_p.reshape(NS, nch, CHUNK)
    dst_r = dst_p.reshape(NS, nch, CHUNK)

    degp = _sc_degree(dst_r, npad=npad, nch=nch)

    b2d = b_gcn.reshape(1, hid)
    xs_t0, nrm = _tc_project_first(feat_list[0:1], W_gcn, b2d, degp, br=br)
    parts = [_sc_scatter(xs_t0, src_r, dst_r, t=0, npad=npad, nch=nch)]
    xs_rest = _tc_project_rest(feat_list[1:], W_gcn, b2d, nrm, br=br)
    for t in range(1, t_steps):
        parts.append(
            _sc_scatter(xs_rest, src_r, dst_r, t=t - 1, npad=npad, nch=nch))

    return _tc_lstm_head(parts, nrm, Wi.T, Wh.T,
                         (bi + bh).reshape(1, 4 * hid),
                         W1, b1.reshape(1, -1), W2, b2.reshape(1, -1), br=br)


# speedup vs baseline: 18.8787x; 1.0761x over previous
import functools

import jax
import jax.numpy as jnp
from jax import lax
from jax.experimental import pallas as pl
from jax.experimental.pallas import tpu as pltpu
from jax.experimental.pallas import tpu_sc as plsc

NC = 2
NS = 16
CHUNK = 128
FH = 64
NSLOT = 8
LOOK = 4


def _round_up(x, m):
    return (x + m - 1) // m * m


def _sc_degree(dst_r, *, npad, nch):
    mesh = plsc.VectorSubcoreMesh(core_axis_name="c", subcore_axis_name="s")
    rpt = npad // NS
    nstrip = rpt // CHUNK
    nchc = nch // NC

    @functools.partial(
        pl.kernel,
        out_type=jax.ShapeDtypeStruct((NC, npad, 16), jnp.float32),
        mesh=mesh,
        scratch_types=[
            pltpu.VMEM((nchc, CHUNK), jnp.int32),
            pltpu.VMEM((CHUNK, 16), jnp.float32),
            pltpu.VMEM((CHUNK, 16), jnp.float32),
            pltpu.VMEM_SHARED((npad, 16), jnp.float32),
            pltpu.SemaphoreType.DMA,
        ],
        compiler_params=pltpu.CompilerParams(use_tc_tiling_on_sc=False),
    )
    def deg_kernel(dst_hbm, out_hbm, dst_v, ones_v, zero_v, deg_sh, sem):
        cid = lax.axis_index("c")
        sid = lax.axis_index("s")
        base = sid * rpt
        pltpu.sync_copy(dst_hbm.at[sid, pl.ds(cid * nchc, nchc)], dst_v)

        def fill(buf, val):
            def body(j, carry):
                buf[j] = jnp.full((16,), val, jnp.float32)
                return carry
            lax.fori_loop(0, CHUNK, body, 0)

        fill(zero_v, 0.0)
        fill(ones_v, 1.0)

        def zcopy(k, carry):
            pltpu.sync_copy(zero_v, deg_sh.at[pl.ds(base + k * CHUNK, CHUNK)])
            return carry
        lax.fori_loop(0, nstrip, zcopy, 0)
        plsc.subcore_barrier()

        def round8(r, carry):
            for j in range(8):
                pltpu.async_copy(ones_v, deg_sh.at[dst_v.at[r * 8 + j]], sem,
                                 add=True)
            for j in range(8):
                pltpu.make_async_copy(
                    ones_v, deg_sh.at[dst_v.at[r * 8 + j]], sem).wait()
            return carry
        lax.fori_loop(0, nchc // 8, round8, 0)
        plsc.subcore_barrier()

        def out_copy(k, carry):
            sl = pl.ds(base + k * CHUNK, CHUNK)
            pltpu.sync_copy(deg_sh.at[sl], ones_v)
            pltpu.sync_copy(ones_v, out_hbm.at[cid, sl])
            return carry
        lax.fori_loop(0, nstrip, out_copy, 0)

    return deg_kernel(dst_r)


def _sc_scatter(xs_all, src_r, dst_r, *, t, npad, nch):
    mesh = plsc.VectorSubcoreMesh(core_axis_name="c", subcore_axis_name="s")
    rpt = npad // NS
    nstrip = rpt // CHUNK
    hid = xs_all.shape[-1]
    nchc = nch // NC

    @functools.partial(
        pl.kernel,
        out_type=jax.ShapeDtypeStruct((NC, npad, hid), jnp.bfloat16),
        mesh=mesh,
        scratch_types=[
            pltpu.VMEM((nchc, CHUNK), jnp.int32),
            pltpu.VMEM((nchc, CHUNK), jnp.int32),
        ] + [pltpu.VMEM((CHUNK, hid), jnp.bfloat16) for _ in range(NSLOT)] + [
            pltpu.VMEM_SHARED((npad, hid), jnp.bfloat16),
        ] + [pltpu.SemaphoreType.DMA for _ in range(2 * NSLOT)],
        compiler_params=pltpu.CompilerParams(use_tc_tiling_on_sc=False),
    )
    def scat_kernel(xs_hbm, src_hbm, dst_hbm, out_hbm,
                    src_v, dst_v, *rest):
        bufs = rest[:NSLOT]
        agg_sh = rest[NSLOT]
        gsem = rest[NSLOT + 1:NSLOT + 1 + NSLOT]
        ssem = rest[NSLOT + 1 + NSLOT:]
        cid = lax.axis_index("c")
        sid = lax.axis_index("s")
        base = sid * rpt
        my_xs = xs_hbm.at[t]

        pltpu.sync_copy(src_hbm.at[sid, pl.ds(cid * nchc, nchc)], src_v)
        pltpu.sync_copy(dst_hbm.at[sid, pl.ds(cid * nchc, nchc)], dst_v)

        def gath(k, slot):
            pltpu.async_copy(my_xs.at[src_v.at[k]], bufs[slot], gsem[slot])

        def gwait(k, slot):
            pltpu.make_async_copy(
                my_xs.at[src_v.at[k]], bufs[slot], gsem[slot]).wait()

        def scat(k, slot):
            pltpu.async_copy(bufs[slot], agg_sh.at[dst_v.at[k]], ssem[slot],
                             add=True)

        def swait(k, slot):
            pltpu.make_async_copy(
                bufs[slot], agg_sh.at[dst_v.at[k]], ssem[slot]).wait()

        for k in range(LOOK):
            gath(k, k)

        zbuf = bufs[NSLOT - 1]

        def zfill(j, carry):
            r = j // (hid // 32)
            col = j % (hid // 32)
            zbuf[r, pl.ds(col * 32, 32)] = jnp.zeros((32,), jnp.bfloat16)
            return carry
        lax.fori_loop(0, CHUNK * (hid // 32), zfill, 0)

        def zcopy(k, carry):
            pltpu.sync_copy(zbuf, agg_sh.at[pl.ds(base + k * CHUNK, CHUNK)])
            return carry
        lax.fori_loop(0, nstrip, zcopy, 0)
        plsc.subcore_barrier()

        for k in range(LOOK):
            gwait(k, k)
            scat(k, k)
            gath(k + LOOK, k + LOOK)

        def step(k, slot):
            gwait(k, slot)
            scat(k, slot)
            old = (slot + LOOK) % NSLOT
            swait(k - LOOK, old)
            gath(k + LOOK, old)

        def ring(i, carry):
            kb = LOOK + i * NSLOT
            for j in range(NSLOT):
                step(kb + j, (LOOK + j) % NSLOT)
            return carry
        lax.fori_loop(0, (nchc - 2 * LOOK) // NSLOT, ring, 0)

        for k in range(nchc - LOOK, nchc):
            slot = k % NSLOT
            gwait(k, slot)
            scat(k, slot)
            swait(k - LOOK, (slot + LOOK) % NSLOT)
        for k in range(nchc - LOOK, nchc):
            swait(k, k % NSLOT)
        plsc.subcore_barrier()

        def out_copy(k, carry):
            sl = pl.ds(base + k * CHUNK, CHUNK)
            pltpu.sync_copy(agg_sh.at[sl], bufs[0])
            pltpu.sync_copy(bufs[0], out_hbm.at[cid, sl])
            return carry
        lax.fori_loop(0, nstrip, out_copy, 0)

    return scat_kernel(xs_all, src_r, dst_r)


def _proj_first_body(f_ref, w_ref, b_ref, d0_ref, d1_ref, xs_ref, norm_ref):
    deg = d0_ref[0][:, 0:1] + d1_ref[0][:, 0:1]
    nrm = lax.rsqrt(jnp.clip(deg, 1.0, None))
    x = jnp.dot(f_ref[0], w_ref[...], preferred_element_type=jnp.float32)
    xs_ref[0] = ((x + b_ref[...]) * nrm).astype(jnp.bfloat16)
    norm_ref[...] = nrm


def _tc_project_first(feat, w, b2d, degp, *, br):
    t, n, din = feat.shape
    hid = w.shape[1]
    nblk = n // br
    return pl.pallas_call(
        _proj_first_body,
        grid=(t, nblk),
        in_specs=[
            pl.BlockSpec((1, br, din), lambda i, r: (i, r, 0)),
            pl.BlockSpec((din, hid), lambda i, r: (0, 0)),
            pl.BlockSpec((1, hid), lambda i, r: (0, 0)),
            pl.BlockSpec((1, br, 16), lambda i, r: (0, r, 0)),
            pl.BlockSpec((1, br, 16), lambda i, r: (1, r, 0)),
        ],
        out_specs=[
            pl.BlockSpec((1, br, hid), lambda i, r: (i, r, 0)),
            pl.BlockSpec((br, 1), lambda i, r: (r, 0)),
        ],
        out_shape=[
            jax.ShapeDtypeStruct((t, n, hid), jnp.bfloat16),
            jax.ShapeDtypeStruct((n, 1), jnp.float32),
        ],
    )(feat, w, b2d, degp, degp)


def _proj_rest_body(f_ref, w_ref, b_ref, nrm_ref, xs_ref):
    x = jnp.dot(f_ref[0], w_ref[...], preferred_element_type=jnp.float32)
    xs_ref[0] = ((x + b_ref[...]) * nrm_ref[...]).astype(jnp.bfloat16)


def _tc_project_rest(feat, w, b2d, nrm, *, br):
    t, n, din = feat.shape
    hid = w.shape[1]
    nblk = n // br
    return pl.pallas_call(
        _proj_rest_body,
        grid=(t, nblk),
        in_specs=[
            pl.BlockSpec((1, br, din), lambda i, r: (i, r, 0)),
            pl.BlockSpec((din, hid), lambda i, r: (0, 0)),
            pl.BlockSpec((1, hid), lambda i, r: (0, 0)),
            pl.BlockSpec((br, 1), lambda i, r: (r, 0)),
        ],
        out_specs=pl.BlockSpec((1, br, hid), lambda i, r: (i, r, 0)),
        out_shape=jax.ShapeDtypeStruct((t, n, hid), jnp.bfloat16),
    )(feat, w, b2d, nrm)


def _lstm_head_body(nrm_ref, wi_ref, wh_ref, bs_ref, w1_ref, b1_ref,
                    w2_ref, b2_ref, *rest):
    part_refs = rest[:-1]
    o_ref = rest[-1]
    hid = wh_ref.shape[0]
    nrm = nrm_ref[...]
    h = None
    c = None
    for p_ref in part_refs:
        agg = (p_ref[0].astype(jnp.float32)
               + p_ref[1].astype(jnp.float32))
        g = jnp.maximum(agg * nrm, 0.0)
        gates = jnp.dot(g, wi_ref[...], preferred_element_type=jnp.float32)
        if h is not None:
            gates = gates + jnp.dot(h, wh_ref[...],
                                    preferred_element_type=jnp.float32)
        gates = gates + bs_ref[...]
        i_g = jax.nn.sigmoid(gates[:, 0:hid])
        f_g = jax.nn.sigmoid(gates[:, hid:2 * hid])
        g_g = jnp.tanh(gates[:, 2 * hid:3 * hid])
        o_g = jax.nn.sigmoid(gates[:, 3 * hid:4 * hid])
        c = i_g * g_g if c is None else f_g * c + i_g * g_g
        h = o_g * jnp.tanh(c)
    z = jnp.maximum(
        jnp.dot(h, w1_ref[...], preferred_element_type=jnp.float32)
        + b1_ref[...], 0.0)
    o_ref[...] = (jnp.dot(z, w2_ref[...], preferred_element_type=jnp.float32)
                  + b2_ref[...])


def _tc_lstm_head(parts, nrm, wi_t, wh_t, bsum, w1, b1_2d, w2, b2_2d, *, br):
    n = nrm.shape[0]
    hid = wh_t.shape[0]
    cls = w1.shape[1]
    dout = w2.shape[1]
    nblk = n // br
    return pl.pallas_call(
        _lstm_head_body,
        grid=(nblk,),
        in_specs=[
            pl.BlockSpec((br, 1), lambda r: (r, 0)),
            pl.BlockSpec((hid, 4 * hid), lambda r: (0, 0)),
            pl.BlockSpec((hid, 4 * hid), lambda r: (0, 0)),
            pl.BlockSpec((1, 4 * hid), lambda r: (0, 0)),
            pl.BlockSpec((hid, cls), lambda r: (0, 0)),
            pl.BlockSpec((1, cls), lambda r: (0, 0)),
            pl.BlockSpec((cls, dout), lambda r: (0, 0)),
            pl.BlockSpec((1, dout), lambda r: (0, 0)),
        ] + [pl.BlockSpec((2, br, hid), lambda r: (0, r, 0))
             for _ in parts],
        out_specs=pl.BlockSpec((br, dout), lambda r: (r, 0)),
        out_shape=jax.ShapeDtypeStruct((n, dout), jnp.float32),
    )(nrm, wi_t, wh_t, bsum, w1, b1_2d, w2, b2_2d, *parts)


def kernel(feat_list, edge_index, n_step, W_gcn, b_gcn, Wi, Wh, bi, bh,
           W1, b1, W2, b2):
    del n_step
    t_steps, n, _ = feat_list.shape
    hid = W_gcn.shape[1]
    br = 1000 if n % 1000 == 0 else 8 * (n // 8)

    src = edge_index[0]
    dst = edge_index[1]
    e = src.shape[0]

    npad = _round_up(n + 1, NS * CHUNK)
    epad = _round_up(e, NS * CHUNK * NSLOT * NC)
    nch = epad // (NS * CHUNK)
    pad = epad - e
    dmask = 1
    while dmask * 2 <= npad - n:
        dmask *= 2
    smask = 1
    while smask * 2 <= n:
        smask *= 2
    pad_i = jnp.arange(pad, dtype=jnp.int32)
    src_p = jnp.concatenate([src, pad_i & (smask - 1)])
    dst_p = jnp.concatenate([dst, n + (pad_i & (dmask - 1))])
    src_r = src_p.reshape(NS, nch, CHUNK)
    dst_r = dst_p.reshape(NS, nch, CHUNK)

    degp = _sc_degree(dst_r, npad=npad, nch=nch)

    b2d = b_gcn.reshape(1, hid)
    xs_t0, nrm = _tc_project_first(feat_list[0:1], W_gcn, b2d, degp, br=br)
    parts = [_sc_scatter(xs_t0, src_r, dst_r, t=0, npad=npad, nch=nch)]
    xs_rest = _tc_project_rest(feat_list[1:], W_gcn, b2d, nrm, br=br)
    for t in range(1, t_steps):
        parts.append(
            _sc_scatter(xs_rest, src_r, dst_r, t=t - 1, npad=npad, nch=nch))

    return _tc_lstm_head(parts, nrm, Wi.T, Wh.T,
                         (bi + bh).reshape(1, 4 * hid),
                         W1, b1.reshape(1, -1), W2, b2.reshape(1, -1), br=br)
